# baseline (speedup 1.0000x reference)
import functools

import jax
import jax.numpy as jnp
from jax import lax
from jax.experimental import pallas as pl
from jax.experimental.pallas import tpu as pltpu
from jax.experimental.pallas import tpu_sc as plsc

BN_EPS = 1e-5

_CHUNK = 128
_CPB = 10
_NPAD = 8


def _make_sc_aggregate(N, D, nblk):
    info = plsc.get_sparse_core_info()
    NC, NS = info.num_cores, info.num_subcores
    chunk, cpb = _CHUNK, _CPB
    rps = (N // NS) // 8 * 8
    rem = N - NS * rps
    assert rem % 8 == 0

    mesh = plsc.VectorSubcoreMesh(core_axis_name="c", subcore_axis_name="s")

    @functools.partial(
        pl.kernel,
        mesh=mesh,
        out_type=jax.ShapeDtypeStruct((NC, N, D), jnp.float32),
        scratch_types=[
            pltpu.VMEM((cpb, 2, chunk), jnp.int32),
            pltpu.VMEM((chunk, D), jnp.float32),
            pltpu.VMEM((chunk, D), jnp.float32),
            pltpu.VMEM_SHARED((N + _NPAD, D), jnp.float32),
            pltpu.SemaphoreType.DMA,
            pltpu.SemaphoreType.DMA,
            pltpu.SemaphoreType.DMA,
            pltpu.SemaphoreType.DMA,
        ],
    )
    def agg_kernel(x_hbm, ei_hbm, zero_hbm, out_hbm,
                   idx, rows0, rows1, acc, gsem0, gsem1, ssem0, ssem1):
        c = lax.axis_index("c")
        s = lax.axis_index("s")
        wid = s * NC + c

        @pl.when(c == 0)
        def _():
            pltpu.sync_copy(x_hbm.at[pl.ds(s * rps, rps)],
                            acc.at[pl.ds(s * rps, rps)])

            @pl.when(s == NS - 1)
            def _():
                pltpu.sync_copy(x_hbm.at[pl.ds(NS * rps, rem)],
                                acc.at[pl.ds(NS * rps, rem)])

        @pl.when(c != 0)
        def _():
            pltpu.sync_copy(zero_hbm.at[pl.ds(s * rps, rps)],
                            acc.at[pl.ds(s * rps, rps)])

            @pl.when(s == NS - 1)
            def _():
                pltpu.sync_copy(zero_hbm.at[pl.ds(NS * rps, rem)],
                                acc.at[pl.ds(NS * rps, rem)])

        plsc.subcore_barrier()

        def gather(j, rows, sem):
            return pltpu.async_copy(x_hbm.at[idx.at[j, 0]], rows, sem)

        def gather_wait(j, rows, sem):
            pltpu.make_async_copy(x_hbm.at[idx.at[j, 0]], rows, sem).wait()

        def scat(j, rows, sem):
            return pltpu.async_copy(rows, acc.at[idx.at[j, 1]], sem,
                                    add=True)

        def scat_wait(j, rows, sem):
            pltpu.make_async_copy(rows, acc.at[idx.at[j, 1]], sem).wait()

        def blk_body(blk, carry0):
            pltpu.sync_copy(ei_hbm.at[wid, blk], idx)
            gather(0, rows0, gsem0)
            gather(1, rows1, gsem1)

            def pair_body(i, carry):
                a = 2 * i
                gather_wait(a, rows0, gsem0)
                scat(a, rows0, ssem0)
                gather_wait(a + 1, rows1, gsem1)
                scat(a + 1, rows1, ssem1)
                scat_wait(a, rows0, ssem0)
                gather(a + 2, rows0, gsem0)
                scat_wait(a + 1, rows1, ssem1)
                gather(a + 3, rows1, gsem1)
                return carry

            lax.fori_loop(0, (cpb - 2) // 2, pair_body, 0)
            gather_wait(cpb - 2, rows0, gsem0)
            scat(cpb - 2, rows0, ssem0)
            gather_wait(cpb - 1, rows1, gsem1)
            scat(cpb - 1, rows1, ssem1)
            scat_wait(cpb - 2, rows0, ssem0)
            scat_wait(cpb - 1, rows1, ssem1)
            return carry0

        lax.fori_loop(0, nblk, blk_body, 0)

        plsc.subcore_barrier()

        pltpu.sync_copy(acc.at[pl.ds(s * rps, rps)],
                        out_hbm.at[c, pl.ds(s * rps, rps)])

        @pl.when(s == NS - 1)
        def _():
            pltpu.sync_copy(acc.at[pl.ds(NS * rps, rem)],
                            out_hbm.at[c, pl.ds(NS * rps, rem)])

    return agg_kernel


def _mlp_bn_body(agg_ref, w1_ref, b1_ref, w2_ref, b2_ref,
                 g_ref, beta_ref, o_ref):
    h = agg_ref[0] + agg_ref[1]
    h = jnp.dot(h, w1_ref[...], preferred_element_type=jnp.float32)
    h = jnp.maximum(h + b1_ref[...], 0.0)
    h = jnp.dot(h, w2_ref[...], preferred_element_type=jnp.float32)
    h = jnp.maximum(h + b2_ref[...], 0.0)
    mean = jnp.mean(h, axis=0, keepdims=True)
    d = h - mean
    var = jnp.mean(d * d, axis=0, keepdims=True)
    o_ref[...] = g_ref[...] * d * lax.rsqrt(var + BN_EPS) + beta_ref[...]


def kernel(x, W1, b1, W2, b2, gamma, beta, edge_index):
    N, D = x.shape
    H = W1.shape[1]
    E = edge_index.shape[1]

    info = plsc.get_sparse_core_info()
    NW = info.num_cores * info.num_subcores
    epb = _CHUNK * _CPB
    epw = -(-E // (NW * epb)) * epb
    e_pad = NW * epw - E
    nblk = epw // epb

    src = jnp.concatenate(
        [edge_index[0], jnp.zeros((e_pad,), jnp.int32)])
    dst = jnp.concatenate(
        [edge_index[1], jnp.full((e_pad,), N, jnp.int32)])
    ei = jnp.stack([src.reshape(NW, nblk, _CPB, _CHUNK),
                    dst.reshape(NW, nblk, _CPB, _CHUNK)], axis=3)
    zeros = jnp.zeros((N, D), dtype=jnp.float32)

    agg2 = _make_sc_aggregate(N, D, nblk)(x, ei, zeros)

    out = pl.pallas_call(
        _mlp_bn_body,
        out_shape=jax.ShapeDtypeStruct((N, H), jnp.float32),
    )(agg2, W1, b1.reshape(1, H), W2, b2.reshape(1, H),
      gamma.reshape(1, H), beta.reshape(1, H))
    return out


# speedup vs baseline: 1.0281x; 1.0281x over previous
import functools

import jax
import jax.numpy as jnp
from jax import lax
from jax.experimental import pallas as pl
from jax.experimental.pallas import tpu as pltpu
from jax.experimental.pallas import tpu_sc as plsc

BN_EPS = 1e-5

_CHUNK = 128
_CPB = 10
_NPAD = 8


def _make_sc_aggregate(N, D, nblk):
    info = plsc.get_sparse_core_info()
    NC, NS = info.num_cores, info.num_subcores
    chunk, cpb = _CHUNK, _CPB
    rps = (N // NS) // 8 * 8
    rem = N - NS * rps
    assert rem % 8 == 0

    mesh = plsc.VectorSubcoreMesh(core_axis_name="c", subcore_axis_name="s")

    @functools.partial(
        pl.kernel,
        mesh=mesh,
        out_type=jax.ShapeDtypeStruct((NC, N, D), jnp.float32),
        scratch_types=[
            pltpu.VMEM((cpb, 2, chunk), jnp.int32),
            pltpu.VMEM((chunk, D), jnp.float32),
            pltpu.VMEM((chunk, D), jnp.float32),
            pltpu.VMEM_SHARED((N + _NPAD, D), jnp.float32),
            pltpu.SemaphoreType.DMA,
            pltpu.SemaphoreType.DMA,
        ],
    )
    def agg_kernel(x_hbm, ei_hbm, zero_hbm, out_hbm,
                   idx, rows0, rows1, acc, gsem0, gsem1):
        c = lax.axis_index("c")
        s = lax.axis_index("s")
        wid = s * NC + c

        @pl.when(c == 0)
        def _():
            pltpu.sync_copy(x_hbm.at[pl.ds(s * rps, rps)],
                            acc.at[pl.ds(s * rps, rps)])

            @pl.when(s == NS - 1)
            def _():
                pltpu.sync_copy(x_hbm.at[pl.ds(NS * rps, rem)],
                                acc.at[pl.ds(NS * rps, rem)])

        @pl.when(c != 0)
        def _():
            pltpu.sync_copy(zero_hbm.at[pl.ds(s * rps, rps)],
                            acc.at[pl.ds(s * rps, rps)])

            @pl.when(s == NS - 1)
            def _():
                pltpu.sync_copy(zero_hbm.at[pl.ds(NS * rps, rem)],
                                acc.at[pl.ds(NS * rps, rem)])

        plsc.subcore_barrier()

        def gather(j, rows, sem):
            return pltpu.async_copy(x_hbm.at[idx.at[j, 0]], rows, sem)

        def gather_wait(j, rows, sem):
            pltpu.make_async_copy(x_hbm.at[idx.at[j, 0]], rows, sem).wait()

        def scat(j, rows):
            pltpu.sync_copy(rows, acc.at[idx.at[j, 1]], add=True)

        def blk_body(blk, carry0):
            pltpu.sync_copy(ei_hbm.at[wid, blk], idx)
            gather(0, rows0, gsem0)
            gather(1, rows1, gsem1)

            def pair_body(i, carry):
                a = 2 * i
                gather_wait(a, rows0, gsem0)
                scat(a, rows0)
                gather(a + 2, rows0, gsem0)
                gather_wait(a + 1, rows1, gsem1)
                scat(a + 1, rows1)
                gather(a + 3, rows1, gsem1)
                return carry

            lax.fori_loop(0, (cpb - 2) // 2, pair_body, 0)
            gather_wait(cpb - 2, rows0, gsem0)
            scat(cpb - 2, rows0)
            gather_wait(cpb - 1, rows1, gsem1)
            scat(cpb - 1, rows1)
            return carry0

        lax.fori_loop(0, nblk, blk_body, 0)

        plsc.subcore_barrier()

        pltpu.sync_copy(acc.at[pl.ds(s * rps, rps)],
                        out_hbm.at[c, pl.ds(s * rps, rps)])

        @pl.when(s == NS - 1)
        def _():
            pltpu.sync_copy(acc.at[pl.ds(NS * rps, rem)],
                            out_hbm.at[c, pl.ds(NS * rps, rem)])

    return agg_kernel


def _mlp_bn_body(agg_ref, w1_ref, b1_ref, w2_ref, b2_ref,
                 g_ref, beta_ref, o_ref):
    h = agg_ref[0] + agg_ref[1]
    h = jnp.dot(h, w1_ref[...], preferred_element_type=jnp.float32)
    h = jnp.maximum(h + b1_ref[...], 0.0)
    h = jnp.dot(h, w2_ref[...], preferred_element_type=jnp.float32)
    h = jnp.maximum(h + b2_ref[...], 0.0)
    mean = jnp.mean(h, axis=0, keepdims=True)
    d = h - mean
    var = jnp.mean(d * d, axis=0, keepdims=True)
    o_ref[...] = g_ref[...] * d * lax.rsqrt(var + BN_EPS) + beta_ref[...]


def kernel(x, W1, b1, W2, b2, gamma, beta, edge_index):
    N, D = x.shape
    H = W1.shape[1]
    E = edge_index.shape[1]

    info = plsc.get_sparse_core_info()
    NW = info.num_cores * info.num_subcores
    epb = _CHUNK * _CPB
    epw = -(-E // (NW * epb)) * epb
    e_pad = NW * epw - E
    nblk = epw // epb

    src = jnp.concatenate(
        [edge_index[0], jnp.zeros((e_pad,), jnp.int32)])
    dst = jnp.concatenate(
        [edge_index[1], jnp.full((e_pad,), N, jnp.int32)])
    ei = jnp.stack([src.reshape(NW, nblk, _CPB, _CHUNK),
                    dst.reshape(NW, nblk, _CPB, _CHUNK)], axis=3)
    zeros = jnp.zeros((N, D), dtype=jnp.float32)

    agg2 = _make_sc_aggregate(N, D, nblk)(x, ei, zeros)

    out = pl.pallas_call(
        _mlp_bn_body,
        out_shape=jax.ShapeDtypeStruct((N, H), jnp.float32),
    )(agg2, W1, b1.reshape(1, H), W2, b2.reshape(1, H),
      gamma.reshape(1, H), beta.reshape(1, H))
    return out
